# baseline (speedup 1.0000x reference)
import functools

import jax
import jax.numpy as jnp
from jax import lax
from jax.experimental import pallas as pl
from jax.experimental.pallas import tpu as pltpu
from jax.experimental.pallas import tpu_sc as plsc

BI = 512


def _select_body(r_ref, colx, coly, colz, rowx, rowy, rowz, xyzt_ref,
                 idx_ref, gidx_ref, gxyz_ref):
    b = pl.program_id(0)
    ib = pl.program_id(1)
    r = r_ref[0, 0]
    n = colx.shape[-2]
    bi = rowx.shape[-1]

    xj = colx[0]
    yj = coly[0]
    zj = colz[0]
    xi = rowx[0]
    yi = rowy[0]
    zi = rowz[0]

    dx = xj - xi
    dy = yj - yi
    dz = zj - zi
    dist2 = dx * dx + dy * dy + dz * dz

    jota = lax.broadcasted_iota(jnp.int32, (n, bi), 0)
    iglob = lax.broadcasted_iota(jnp.int32, (n, bi), 1) + ib * bi
    within = ((jnp.abs(dx) < r) & (jnp.abs(dy) < r) & (jnp.abs(dz) < r)
              & (jota != iglob))
    inf = jnp.float32(1e10)
    d_in = jnp.where(within, dist2, inf)
    iglob_row = lax.broadcasted_iota(jnp.int32, (1, bi), 1) + ib * bi
    octant = ((dx > 0).astype(jnp.int32) * 4
              + (dy > 0).astype(jnp.int32) * 2
              + (dz > 0).astype(jnp.int32))

    xyzt = xyzt_ref[0]
    h1 = xyzt.astype(jnp.bfloat16)
    r1 = xyzt - h1.astype(jnp.float32)
    h2 = r1.astype(jnp.bfloat16)
    h3 = (r1 - h2.astype(jnp.float32)).astype(jnp.bfloat16)
    jrow_f = lax.broadcasted_iota(jnp.int32, (1, n), 1).astype(jnp.float32)
    j1 = jrow_f.astype(jnp.bfloat16)
    j2 = (jrow_f - j1.astype(jnp.float32)).astype(jnp.bfloat16)
    ones = jnp.full((1, n), 1.0, jnp.bfloat16)
    hm = jnp.concatenate([h1, h2, h3, j1, j2, ones], axis=0)
    center = jnp.concatenate([xi, yi, zi], axis=0)
    dn = (((1,), (0,)), ((), ()))

    for o in range(8):
        d_o = jnp.where(octant == o, d_in, inf)
        mn = jnp.min(d_o, axis=0, keepdims=True)
        valid = mn < 1e9
        q = (d_o == mn) & valid
        qb = q.astype(jnp.bfloat16)
        s = jax.lax.dot_general(hm, qb, dn,
                                preferred_element_type=jnp.float32)
        g = s[0:3] + s[3:6] + s[6:9]
        cand = (s[9:10] + s[10:11]).astype(jnp.int32)
        cnt = s[11:12]
        sel = jnp.where(valid, cand, iglob_row)
        idx_ref[0, o:o + 1, :] = sel
        gidx_ref[0, o:o + 1, :] = sel + b * n
        gxyz_ref[0, 3 * o:3 * o + 3, :] = jnp.where(valid, g - center, 0.0)

        tie = jnp.max(cnt) > 1.5

        @pl.when(tie)
        def _fix(o=o, q=q, valid=valid):
            jf = jota.astype(jnp.float32)
            cand2 = jnp.min(jnp.where(q, jf, jnp.float32(n)),
                            axis=0, keepdims=True).astype(jnp.int32)
            sel2 = jnp.where(valid, cand2, iglob_row)
            idx_ref[0, o:o + 1, :] = sel2
            gidx_ref[0, o:o + 1, :] = sel2 + b * n
            oh2 = (jota == sel2).astype(jnp.bfloat16)
            s2 = jax.lax.dot_general(hm, oh2, dn,
                                     preferred_element_type=jnp.float32)
            g2 = s2[0:3] + s2[3:6] + s2[6:9]
            gxyz_ref[0, 3 * o:3 * o + 3, :] = jnp.where(valid, g2 - center, 0.0)


def _select_cube_tc(xyz, radius):
    B, N, _ = xyz.shape
    x = xyz[:, :, 0]
    y = xyz[:, :, 1]
    z = xyz[:, :, 2]
    col = lambda a: a[:, :, None]
    row = lambda a: a[:, None, :]
    xyzt = jnp.stack([x, y, z], axis=1)
    r2 = jnp.reshape(radius, (1, 1))

    col_spec = pl.BlockSpec((1, N, 1), lambda b, i: (b, 0, 0))
    row_spec = pl.BlockSpec((1, 1, BI), lambda b, i: (b, 0, i))
    grid = (B, N // BI)
    idx_t, gidx_t, gxyz_t = pl.pallas_call(
        _select_body,
        grid=grid,
        in_specs=[
            pl.BlockSpec(memory_space=pltpu.SMEM),
            col_spec, col_spec, col_spec,
            row_spec, row_spec, row_spec,
            pl.BlockSpec((1, 3, N), lambda b, i: (b, 0, 0)),
        ],
        out_specs=[
            pl.BlockSpec((1, 8, BI), lambda b, i: (b, 0, i)),
            pl.BlockSpec((1, 8, BI), lambda b, i: (b, 0, i)),
            pl.BlockSpec((1, 24, BI), lambda b, i: (b, 0, i)),
        ],
        out_shape=[
            jax.ShapeDtypeStruct((B, 8, N), jnp.int32),
            jax.ShapeDtypeStruct((B, 8, N), jnp.int32),
            jax.ShapeDtypeStruct((B, 24, N), jnp.float32),
        ],
    )(r2, col(x), col(y), col(z), row(x), row(y), row(z), xyzt)
    return idx_t, gidx_t, gxyz_t


def _sc_gather(table, idxs):
    R, D = table.shape
    M = idxs.shape[0]
    info = plsc.get_sparse_core_info()
    nw = info.num_cores * info.num_subcores
    per_w = M // nw
    CH = 128
    T = per_w // CH
    idx3 = idxs.reshape(nw, T, CH)
    mesh = plsc.VectorSubcoreMesh(core_axis_name="c", subcore_axis_name="s")

    @functools.partial(
        pl.kernel,
        mesh=mesh,
        compiler_params=pltpu.CompilerParams(use_tc_tiling_on_sc=False),
        out_type=jax.ShapeDtypeStruct((M, D), jnp.float32),
        scratch_types=[
            pltpu.VMEM((T, CH), jnp.int32),
            pltpu.VMEM((CH, D), jnp.float32),
            pltpu.VMEM((CH, D), jnp.float32),
            pltpu.SemaphoreType.DMA,
            pltpu.SemaphoreType.DMA,
        ],
    )
    def k(idx_hbm, table_hbm, out_hbm, idx_v, rows0, rows1, sem0, sem1):
        wid = lax.axis_index("s") * info.num_cores + lax.axis_index("c")
        pltpu.sync_copy(idx_hbm.at[wid], idx_v)
        bufs = (rows0, rows1)
        sems = (sem0, sem1)
        cps = [None, None]
        cps[0] = pltpu.async_copy(table_hbm.at[idx_v.at[0]], bufs[0], sems[0])
        for t in range(T):
            cur = t % 2
            nxt = (t + 1) % 2
            if t + 1 < T:
                cps[nxt] = pltpu.async_copy(
                    table_hbm.at[idx_v.at[t + 1]], bufs[nxt], sems[nxt])
            cps[cur].wait()
            pltpu.sync_copy(bufs[cur], out_hbm.at[pl.ds(wid * per_w + t * CH, CH)])

    return k(idx3, table)


def kernel(xyz, points, radius):
    B, N, _ = xyz.shape
    P = points.shape[-1]
    idx_t, gidx_t, gxyz_t = _select_cube_tc(xyz, radius)
    idx = jnp.transpose(idx_t, (0, 2, 1))
    gidx = jnp.transpose(gidx_t, (0, 2, 1)).reshape(B * N * 8)
    grouped_xyz = jnp.transpose(gxyz_t, (0, 2, 1)).reshape(B, N, 8, 3)
    gp = _sc_gather(points.reshape(B * N, P), gidx)
    grouped_points = jnp.concatenate(
        [grouped_xyz, gp.reshape(B, N, 8, P)], axis=-1)
    return grouped_xyz, grouped_points, idx


# speedup vs baseline: 1.3404x; 1.3404x over previous
import functools

import jax
import jax.numpy as jnp
from jax import lax
from jax.experimental import pallas as pl
from jax.experimental.pallas import tpu as pltpu
from jax.experimental.pallas import tpu_sc as plsc

BI = 512


def _select_body(r_ref, colx, coly, colz, rowx, rowy, rowz, xyzt_ref,
                 idx_ref, gidx_ref, gxyz_ref):
    b = pl.program_id(0)
    ib = pl.program_id(1)
    r = r_ref[0, 0]
    n = colx.shape[-2]
    bi = rowx.shape[-1]

    xj = colx[0]
    yj = coly[0]
    zj = colz[0]
    xi = rowx[0]
    yi = rowy[0]
    zi = rowz[0]

    dx = xj - xi
    dy = yj - yi
    dz = zj - zi
    dist2 = dx * dx + dy * dy + dz * dz

    jota = lax.broadcasted_iota(jnp.int32, (n, bi), 0)
    iglob = lax.broadcasted_iota(jnp.int32, (n, bi), 1) + ib * bi
    within = ((jnp.abs(dx) < r) & (jnp.abs(dy) < r) & (jnp.abs(dz) < r)
              & (jota != iglob))
    inf = jnp.float32(1e10)
    d_in = jnp.where(within, dist2, inf)
    iglob_row = lax.broadcasted_iota(jnp.int32, (1, bi), 1) + ib * bi
    octant = ((dx > 0).astype(jnp.int32) * 4
              + (dy > 0).astype(jnp.int32) * 2
              + (dz > 0).astype(jnp.int32))

    xyzt = xyzt_ref[0]
    h1 = xyzt.astype(jnp.bfloat16)
    r1 = xyzt - h1.astype(jnp.float32)
    h2 = r1.astype(jnp.bfloat16)
    h3 = (r1 - h2.astype(jnp.float32)).astype(jnp.bfloat16)
    hm = jnp.concatenate([h1, h2, h3], axis=0)
    center = jnp.concatenate([xi, yi, zi], axis=0)
    jota_f = jota.astype(jnp.float32)
    dn = (((1,), (0,)), ((), ()))

    for o in range(8):
        d_o = jnp.where(octant == o, d_in, inf)
        mn = jnp.min(d_o, axis=0, keepdims=True)
        cand_f = jnp.min(jnp.where(d_o == mn, jota_f, jnp.float32(n)),
                         axis=0, keepdims=True)
        cand = cand_f.astype(jnp.int32)
        sel = jnp.where(mn < 1e9, cand, iglob_row)
        idx_ref[0, o:o + 1, :] = sel
        gidx_ref[0, o:o + 1, :] = sel + b * n
        oh = (jota == sel).astype(jnp.bfloat16)
        s = jax.lax.dot_general(hm, oh, dn,
                                preferred_element_type=jnp.float32)
        g = s[0:3] + s[3:6] + s[6:9]
        gxyz_ref[0, 3 * o:3 * o + 3, :] = g - center


def _select_cube_tc(xyz, radius):
    B, N, _ = xyz.shape
    x = xyz[:, :, 0]
    y = xyz[:, :, 1]
    z = xyz[:, :, 2]
    col = lambda a: a[:, :, None]
    row = lambda a: a[:, None, :]
    xyzt = jnp.stack([x, y, z], axis=1)
    r2 = jnp.reshape(radius, (1, 1))

    col_spec = pl.BlockSpec((1, N, 1), lambda b, i: (b, 0, 0))
    row_spec = pl.BlockSpec((1, 1, BI), lambda b, i: (b, 0, i))
    grid = (B, N // BI)
    idx_t, gidx_t, gxyz_t = pl.pallas_call(
        _select_body,
        grid=grid,
        in_specs=[
            pl.BlockSpec(memory_space=pltpu.SMEM),
            col_spec, col_spec, col_spec,
            row_spec, row_spec, row_spec,
            pl.BlockSpec((1, 3, N), lambda b, i: (b, 0, 0)),
        ],
        out_specs=[
            pl.BlockSpec((1, 8, BI), lambda b, i: (b, 0, i)),
            pl.BlockSpec((1, 8, BI), lambda b, i: (b, 0, i)),
            pl.BlockSpec((1, 24, BI), lambda b, i: (b, 0, i)),
        ],
        out_shape=[
            jax.ShapeDtypeStruct((B, 8, N), jnp.int32),
            jax.ShapeDtypeStruct((B, 8, N), jnp.int32),
            jax.ShapeDtypeStruct((B, 24, N), jnp.float32),
        ],
    )(r2, col(x), col(y), col(z), row(x), row(y), row(z), xyzt)
    return idx_t, gidx_t, gxyz_t


def _sc_gather(table, idxs):
    R, D = table.shape
    M = idxs.shape[0]
    info = plsc.get_sparse_core_info()
    nw = info.num_cores * info.num_subcores
    per_w = M // nw
    CH = 128
    T = per_w // CH
    idx3 = idxs.reshape(nw, T, CH)
    mesh = plsc.VectorSubcoreMesh(core_axis_name="c", subcore_axis_name="s")

    @functools.partial(
        pl.kernel,
        mesh=mesh,
        compiler_params=pltpu.CompilerParams(use_tc_tiling_on_sc=False),
        out_type=jax.ShapeDtypeStruct((M, D), jnp.float32),
        scratch_types=[
            pltpu.VMEM((T, CH), jnp.int32),
            pltpu.VMEM((CH, D), jnp.float32),
            pltpu.VMEM((CH, D), jnp.float32),
            pltpu.SemaphoreType.DMA,
            pltpu.SemaphoreType.DMA,
        ],
    )
    def k(idx_hbm, table_hbm, out_hbm, idx_v, rows0, rows1, sem0, sem1):
        wid = lax.axis_index("s") * info.num_cores + lax.axis_index("c")
        pltpu.sync_copy(idx_hbm.at[wid], idx_v)
        bufs = (rows0, rows1)
        sems = (sem0, sem1)
        cps = [None, None]
        cps[0] = pltpu.async_copy(table_hbm.at[idx_v.at[0]], bufs[0], sems[0])
        for t in range(T):
            cur = t % 2
            nxt = (t + 1) % 2
            if t + 1 < T:
                cps[nxt] = pltpu.async_copy(
                    table_hbm.at[idx_v.at[t + 1]], bufs[nxt], sems[nxt])
            cps[cur].wait()
            pltpu.sync_copy(bufs[cur], out_hbm.at[pl.ds(wid * per_w + t * CH, CH)])

    return k(idx3, table)


def kernel(xyz, points, radius):
    B, N, _ = xyz.shape
    P = points.shape[-1]
    idx_t, gidx_t, gxyz_t = _select_cube_tc(xyz, radius)
    idx = jnp.transpose(idx_t, (0, 2, 1))
    gidx = jnp.transpose(gidx_t, (0, 2, 1)).reshape(B * N * 8)
    grouped_xyz = jnp.transpose(gxyz_t, (0, 2, 1)).reshape(B, N, 8, 3)
    gp = _sc_gather(points.reshape(B * N, P), gidx)
    grouped_points = jnp.concatenate(
        [grouped_xyz, gp.reshape(B, N, 8, P)], axis=-1)
    return grouped_xyz, grouped_points, idx
